# baseline (speedup 1.0000x reference)
import functools

import jax
import jax.numpy as jnp
from jax import lax
from jax.experimental import pallas as pl
from jax.experimental.pallas import tpu as pltpu
from jax.experimental.pallas import tpu_sc as plsc

N = 10000
E = 320000
H = 8
C = 16
D = H * C
EN = E + N
NEG = 0.1

Np = 10048
CH = 2048
NCH = 42
EPT = CH * NCH
ENp = 4 * EPT


def _tc_proj(x, A):
    def body(x_ref, A_ref, a_ref, mx_ref):
        a = jnp.dot(x_ref[...], A_ref[...], preferred_element_type=jnp.float32)
        a_ref[...] = a
        mx_ref[...] = jnp.max(a, axis=0, keepdims=True)

    return pl.pallas_call(
        body,
        out_shape=(
            jax.ShapeDtypeStruct((N, 16), jnp.float32),
            jax.ShapeDtypeStruct((1, 16), jnp.float32),
        ),
    )(x, A)


@functools.partial(
    pl.kernel,
    out_type=jax.ShapeDtypeStruct((8, ENp), jnp.float32),
    mesh=plsc.VectorSubcoreMesh(core_axis_name="c", subcore_axis_name="s"),
    compiler_params=pltpu.CompilerParams(needs_layout_passes=False),
    scratch_types=[
        pltpu.VMEM((Np,), jnp.float32),
        pltpu.VMEM((Np,), jnp.float32),
        pltpu.VMEM((Np,), jnp.float32),
        pltpu.VMEM((Np,), jnp.float32),
        pltpu.VMEM((CH,), jnp.int32),
        pltpu.VMEM((CH,), jnp.int32),
        pltpu.VMEM((CH,), jnp.int32),
        pltpu.VMEM((CH,), jnp.int32),
        pltpu.VMEM((CH,), jnp.float32),
        pltpu.VMEM((CH,), jnp.float32),
        pltpu.VMEM((CH,), jnp.float32),
        pltpu.VMEM((CH,), jnp.float32),
        pltpu.VMEM((CH,), jnp.float32),
        pltpu.VMEM((CH,), jnp.float32),
        pltpu.VMEM((16,), jnp.float32),
        pltpu.VMEM_SHARED((16 * Np,), jnp.float32),
        pltpu.SemaphoreType.DMA,
        pltpu.SemaphoreType.DMA,
        pltpu.SemaphoreType.DMA,
        pltpu.SemaphoreType.DMA,
    ],
)
def _sc_attn(aT, m16, znp, srcF, dstF, ewF, out,
             asrc, adst, acc, tmp, ua0, ua1, ub0, ub1, ew0, ew1, p_c0, p_c1,
             ab0, ab1, mbuf, slab, sem_i0, sem_i1, sem_o0, sem_o1):
    c = lax.axis_index("c")
    s = lax.axis_index("s")
    hl = s // 4
    q = s % 4
    h = 4 * c + hl
    ebase = q * EPT

    pltpu.sync_copy(aT.at[h], asrc)
    pltpu.sync_copy(aT.at[8 + h], adst)
    pltpu.sync_copy(znp, acc)
    pltpu.sync_copy(m16, mbuf)
    mv = plsc.load_gather(mbuf, [jnp.full((16,), h, jnp.int32)])
    nvec = Np // 16

    def p1_in_desc(j, sbuf, dbuf, sem):
        off = ebase + j * CH
        return (pltpu.make_async_copy(srcF.at[pl.ds(off, CH)], sbuf, sem),
                pltpu.make_async_copy(dstF.at[pl.ds(off, CH)], dbuf, sem))

    def p1_compute(sbuf, dbuf, pbuf):
        @plsc.parallel_loop(0, CH // 16, unroll=4)
        def _(i):
            sidx = sbuf[pl.ds(i * 16, 16)]
            didx = dbuf[pl.ds(i * 16, 16)]
            e = plsc.load_gather(asrc, [sidx]) + plsc.load_gather(adst, [didx])
            e = jnp.maximum(e, NEG * e)
            p = jnp.exp(e - mv)
            pbuf[pl.ds(i * 16, 16)] = p
            plsc.addupdate_scatter(acc, [didx], p)

    def p1_out_desc(j, pbuf, sem):
        off = ebase + j * CH
        return pltpu.make_async_copy(pbuf, out.at[h, pl.ds(off, CH)], sem)

    def p1_slot(g, j, sbuf, dbuf, pbuf, sem_i, sem_o):
        d1, d2 = p1_in_desc(j, sbuf, dbuf, sem_i)
        d1.wait()
        d2.wait()

        @pl.when(g > 0)
        def _():
            p1_out_desc(j, pbuf, sem_o).wait()

        p1_compute(sbuf, dbuf, pbuf)
        p1_out_desc(j, pbuf, sem_o).start()

        @pl.when(j + 2 < NCH)
        def _():
            n1, n2 = p1_in_desc(j + 2, sbuf, dbuf, sem_i)
            n1.start()
            n2.start()

    d1, d2 = p1_in_desc(0, ua0, ub0, sem_i0)
    d1.start()
    d2.start()
    d1, d2 = p1_in_desc(1, ua1, ub1, sem_i1)
    d1.start()
    d2.start()

    def pair1(g, _):
        p1_slot(g, 2 * g, ua0, ub0, p_c0, sem_i0, sem_o0)
        p1_slot(g, 2 * g + 1, ua1, ub1, p_c1, sem_i1, sem_o1)
        return 0

    lax.fori_loop(0, NCH // 2, pair1, 0)
    p1_out_desc(NCH - 2, p_c0, sem_o0).wait()
    p1_out_desc(NCH - 1, p_c1, sem_o1).wait()

    pltpu.sync_copy(acc, slab.at[pl.ds(s * Np, Np)])
    plsc.subcore_barrier()
    pltpu.sync_copy(slab.at[pl.ds(hl * 4 * Np, Np)], acc)

    def addrow(qq):
        pltpu.sync_copy(slab.at[pl.ds((hl * 4 + qq) * Np, Np)], tmp)

        @plsc.parallel_loop(0, nvec, unroll=8)
        def _(i):
            acc[pl.ds(i * 16, 16)] = acc[pl.ds(i * 16, 16)] + tmp[pl.ds(i * 16, 16)]

    addrow(1)
    addrow(2)
    addrow(3)

    @plsc.parallel_loop(0, nvec, unroll=8)
    def _(i):
        acc[pl.ds(i * 16, 16)] = 1.0 / (acc[pl.ds(i * 16, 16)] + 1e-30)

    def p2_in_desc(j, ebuf, dbuf, pbuf, sem):
        off = ebase + j * CH
        return (pltpu.make_async_copy(ewF.at[pl.ds(off, CH)], ebuf, sem),
                pltpu.make_async_copy(dstF.at[pl.ds(off, CH)], dbuf, sem),
                pltpu.make_async_copy(out.at[h, pl.ds(off, CH)], pbuf, sem))

    def p2_out_desc(j, abuf, sem):
        off = ebase + j * CH
        return pltpu.make_async_copy(abuf, out.at[h, pl.ds(off, CH)], sem)

    def p2_slot(g, j, ebuf, dbuf, pbuf, abuf, sem_i, sem_o):
        for d in p2_in_desc(j, ebuf, dbuf, pbuf, sem_i):
            d.wait()

        @pl.when(g > 0)
        def _():
            p2_out_desc(j, abuf, sem_o).wait()

        @plsc.parallel_loop(0, CH // 16, unroll=4)
        def _(i):
            didx = dbuf[pl.ds(i * 16, 16)]
            inv = plsc.load_gather(acc, [didx])
            sl = pl.ds(i * 16, 16)
            abuf[sl] = pbuf[sl] * ebuf[sl] * inv

        p2_out_desc(j, abuf, sem_o).start()

        @pl.when(j + 2 < NCH)
        def _():
            for d in p2_in_desc(j + 2, ebuf, dbuf, pbuf, sem_i):
                d.start()

    for d in p2_in_desc(0, ew0, ub0, p_c0, sem_i0):
        d.start()
    for d in p2_in_desc(1, ew1, ub1, p_c1, sem_i1):
        d.start()

    def pair2(g, _):
        p2_slot(g, 2 * g, ew0, ub0, p_c0, ab0, sem_i0, sem_o0)
        p2_slot(g, 2 * g + 1, ew1, ub1, p_c1, ab1, sem_i1, sem_o1)
        return 0

    lax.fori_loop(0, NCH // 2, pair2, 0)
    p2_out_desc(NCH - 2, ab0, sem_o0).wait()
    p2_out_desc(NCH - 1, ab1, sem_o1).wait()


def kernel(x, W_src, W_dst, att_src, att_dst, edge_weights, edge_index):
    A_s = jnp.einsum("dhc,hc->dh", W_src.reshape(D, H, C), att_src)
    A_d = jnp.einsum("dhc,hc->dh", W_dst.reshape(D, H, C), att_dst)
    A = jnp.concatenate([A_s, A_d], axis=1)

    a, mx = _tc_proj(x, A)
    ub = mx[0, :8] + mx[0, 8:]
    m8 = jnp.maximum(ub, NEG * ub)
    m16 = jnp.concatenate([m8, m8])

    aT = jnp.zeros((16, Np), jnp.float32).at[:, :N].set(a.T)

    loop = jnp.arange(N, dtype=edge_index.dtype)
    padE = ENp - EN
    dummy = jnp.full((padE,), N, dtype=edge_index.dtype)
    srcF = jnp.concatenate([edge_index[0], loop, dummy])
    dstF = jnp.concatenate([edge_index[1], loop, dummy])
    ewF = jnp.concatenate([
        edge_weights,
        jnp.ones((N,), jnp.float32),
        jnp.zeros((padE,), jnp.float32),
    ])
    znp = jnp.zeros((Np,), jnp.float32)

    alphaT = _sc_attn(aT, m16, znp, srcF, dstF, ewF)
    return alphaT[:, :EN].T


# speedup vs baseline: 1.0501x; 1.0501x over previous
import functools

import jax
import jax.numpy as jnp
from jax import lax
from jax.experimental import pallas as pl
from jax.experimental.pallas import tpu as pltpu
from jax.experimental.pallas import tpu_sc as plsc

N = 10000
E = 320000
H = 8
C = 16
D = H * C
EN = E + N
NEG = 0.1

Np = 10048
CH = 2048
NCH = 42
EPT = CH * NCH
ENp = 4 * EPT


def _tc_proj(x, A):
    def body(x_ref, A_ref, a_ref, mx_ref):
        a = jnp.dot(x_ref[...], A_ref[...], preferred_element_type=jnp.float32)
        a_ref[...] = a
        mx_ref[...] = jnp.max(a, axis=0, keepdims=True)

    return pl.pallas_call(
        body,
        out_shape=(
            jax.ShapeDtypeStruct((N, 16), jnp.float32),
            jax.ShapeDtypeStruct((1, 16), jnp.float32),
        ),
    )(x, A)


@functools.partial(
    pl.kernel,
    out_type=jax.ShapeDtypeStruct((8, ENp), jnp.float32),
    mesh=plsc.VectorSubcoreMesh(core_axis_name="c", subcore_axis_name="s"),
    compiler_params=pltpu.CompilerParams(needs_layout_passes=False),
    scratch_types=[
        pltpu.VMEM((Np,), jnp.float32),
        pltpu.VMEM((Np,), jnp.float32),
        pltpu.VMEM((Np,), jnp.float32),
        pltpu.VMEM((Np,), jnp.float32),
        pltpu.VMEM((CH,), jnp.int32),
        pltpu.VMEM((CH,), jnp.int32),
        pltpu.VMEM((CH,), jnp.int32),
        pltpu.VMEM((CH,), jnp.int32),
        pltpu.VMEM((CH,), jnp.float32),
        pltpu.VMEM((CH,), jnp.float32),
        pltpu.VMEM((CH,), jnp.float32),
        pltpu.VMEM((CH,), jnp.float32),
        pltpu.VMEM((CH,), jnp.float32),
        pltpu.VMEM((CH,), jnp.float32),
        pltpu.VMEM((16,), jnp.float32),
        pltpu.VMEM_SHARED((16 * Np,), jnp.float32),
        pltpu.SemaphoreType.DMA,
        pltpu.SemaphoreType.DMA,
        pltpu.SemaphoreType.DMA,
        pltpu.SemaphoreType.DMA,
    ],
)
def _sc_attn(aT, m16, znp, srcF, dstF, ewF, out,
             asrc, adst, acc, tmp, ua0, ua1, ub0, ub1, ew0, ew1, p_c0, p_c1,
             ab0, ab1, mbuf, slab, sem_i0, sem_i1, sem_o0, sem_o1):
    c = lax.axis_index("c")
    s = lax.axis_index("s")
    hl = s // 4
    q = s % 4
    h = 4 * c + hl
    ebase = q * EPT

    pltpu.sync_copy(aT.at[h], asrc)
    pltpu.sync_copy(aT.at[8 + h], adst)
    pltpu.sync_copy(znp, acc)
    pltpu.sync_copy(m16, mbuf)
    mv = plsc.load_gather(mbuf, [jnp.full((16,), h, jnp.int32)])
    lane = lax.iota(jnp.int32, 16)
    lane15 = lane == 15
    notl15 = lane != 15
    lanep1 = jnp.minimum(lane + 1, 15)
    gdn = lax.GatherDimensionNumbers(
        offset_dims=(), collapsed_slice_dims=(0,), start_index_map=(0,))
    nvec = Np // 16

    def p1_in_desc(j, sbuf, dbuf, sem):
        off = ebase + j * CH
        return (pltpu.make_async_copy(srcF.at[pl.ds(off, CH)], sbuf, sem),
                pltpu.make_async_copy(dstF.at[pl.ds(off, CH)], dbuf, sem))

    def p1_compute(sbuf, dbuf, pbuf):
        @plsc.parallel_loop(0, CH // 16, unroll=8)
        def _(i):
            sidx = sbuf[pl.ds(i * 16, 16)]
            didx = dbuf[pl.ds(i * 16, 16)]
            e = plsc.load_gather(asrc, [sidx]) + plsc.load_gather(adst, [didx])
            e = jnp.maximum(e, NEG * e)
            p = jnp.exp(e - mv)
            pbuf[pl.ds(i * 16, 16)] = p
            ks, vs = plsc.sort_key_val(didx, p)
            nxt = lax.gather(ks, lanep1[:, None], gdn, (1,),
                             mode=lax.GatherScatterMode.PROMISE_IN_BOUNDS)
            mlast = (ks != nxt) | lane15
            css = plsc.cumsum(vs)
            plsc.addupdate_scatter(acc, [ks], css, mask=mlast)
            plsc.addupdate_scatter(acc, [nxt], -css, mask=mlast & notl15)

    def p1_out_desc(j, pbuf, sem):
        off = ebase + j * CH
        return pltpu.make_async_copy(pbuf, out.at[h, pl.ds(off, CH)], sem)

    def p1_slot(g, j, sbuf, dbuf, pbuf, sem_i, sem_o):
        d1, d2 = p1_in_desc(j, sbuf, dbuf, sem_i)
        d1.wait()
        d2.wait()

        @pl.when(g > 0)
        def _():
            p1_out_desc(j, pbuf, sem_o).wait()

        p1_compute(sbuf, dbuf, pbuf)
        p1_out_desc(j, pbuf, sem_o).start()

        @pl.when(j + 2 < NCH)
        def _():
            n1, n2 = p1_in_desc(j + 2, sbuf, dbuf, sem_i)
            n1.start()
            n2.start()

    d1, d2 = p1_in_desc(0, ua0, ub0, sem_i0)
    d1.start()
    d2.start()
    d1, d2 = p1_in_desc(1, ua1, ub1, sem_i1)
    d1.start()
    d2.start()

    def pair1(g, _):
        p1_slot(g, 2 * g, ua0, ub0, p_c0, sem_i0, sem_o0)
        p1_slot(g, 2 * g + 1, ua1, ub1, p_c1, sem_i1, sem_o1)
        return 0

    lax.fori_loop(0, NCH // 2, pair1, 0)
    p1_out_desc(NCH - 2, p_c0, sem_o0).wait()
    p1_out_desc(NCH - 1, p_c1, sem_o1).wait()

    pltpu.sync_copy(acc, slab.at[pl.ds(s * Np, Np)])
    plsc.subcore_barrier()
    pltpu.sync_copy(slab.at[pl.ds(hl * 4 * Np, Np)], acc)

    def addrow(qq):
        pltpu.sync_copy(slab.at[pl.ds((hl * 4 + qq) * Np, Np)], tmp)

        @plsc.parallel_loop(0, nvec, unroll=8)
        def _(i):
            acc[pl.ds(i * 16, 16)] = acc[pl.ds(i * 16, 16)] + tmp[pl.ds(i * 16, 16)]

    addrow(1)
    addrow(2)
    addrow(3)

    @plsc.parallel_loop(0, nvec, unroll=8)
    def _(i):
        acc[pl.ds(i * 16, 16)] = 1.0 / (acc[pl.ds(i * 16, 16)] + 1e-30)

    def p2_in_desc(j, ebuf, dbuf, pbuf, sem):
        off = ebase + j * CH
        return (pltpu.make_async_copy(ewF.at[pl.ds(off, CH)], ebuf, sem),
                pltpu.make_async_copy(dstF.at[pl.ds(off, CH)], dbuf, sem),
                pltpu.make_async_copy(out.at[h, pl.ds(off, CH)], pbuf, sem))

    def p2_out_desc(j, abuf, sem):
        off = ebase + j * CH
        return pltpu.make_async_copy(abuf, out.at[h, pl.ds(off, CH)], sem)

    def p2_slot(g, j, ebuf, dbuf, pbuf, abuf, sem_i, sem_o):
        for d in p2_in_desc(j, ebuf, dbuf, pbuf, sem_i):
            d.wait()

        @pl.when(g > 0)
        def _():
            p2_out_desc(j, abuf, sem_o).wait()

        @plsc.parallel_loop(0, CH // 16, unroll=8)
        def _(i):
            didx = dbuf[pl.ds(i * 16, 16)]
            inv = plsc.load_gather(acc, [didx])
            sl = pl.ds(i * 16, 16)
            abuf[sl] = pbuf[sl] * ebuf[sl] * inv

        p2_out_desc(j, abuf, sem_o).start()

        @pl.when(j + 2 < NCH)
        def _():
            for d in p2_in_desc(j + 2, ebuf, dbuf, pbuf, sem_i):
                d.start()

    for d in p2_in_desc(0, ew0, ub0, p_c0, sem_i0):
        d.start()
    for d in p2_in_desc(1, ew1, ub1, p_c1, sem_i1):
        d.start()

    def pair2(g, _):
        p2_slot(g, 2 * g, ew0, ub0, p_c0, ab0, sem_i0, sem_o0)
        p2_slot(g, 2 * g + 1, ew1, ub1, p_c1, ab1, sem_i1, sem_o1)
        return 0

    lax.fori_loop(0, NCH // 2, pair2, 0)
    p2_out_desc(NCH - 2, ab0, sem_o0).wait()
    p2_out_desc(NCH - 1, ab1, sem_o1).wait()


def kernel(x, W_src, W_dst, att_src, att_dst, edge_weights, edge_index):
    A_s = jnp.einsum("dhc,hc->dh", W_src.reshape(D, H, C), att_src)
    A_d = jnp.einsum("dhc,hc->dh", W_dst.reshape(D, H, C), att_dst)
    A = jnp.concatenate([A_s, A_d], axis=1)

    a, mx = _tc_proj(x, A)
    ub = mx[0, :8] + mx[0, 8:]
    m8 = jnp.maximum(ub, NEG * ub)
    m16 = jnp.concatenate([m8, m8])

    aT = jnp.zeros((16, Np), jnp.float32).at[:, :N].set(a.T)

    loop = jnp.arange(N, dtype=edge_index.dtype)
    padE = ENp - EN
    dummy = jnp.full((padE,), N, dtype=edge_index.dtype)
    srcF = jnp.concatenate([edge_index[0], loop, dummy])
    dstF = jnp.concatenate([edge_index[1], loop, dummy])
    ewF = jnp.concatenate([
        edge_weights,
        jnp.ones((N,), jnp.float32),
        jnp.zeros((padE,), jnp.float32),
    ])
    znp = jnp.zeros((Np,), jnp.float32)

    alphaT = _sc_attn(aT, m16, znp, srcF, dstF, ewF)
    return alphaT[:, :EN].T
